# Initial kernel scaffold; baseline (speedup 1.0000x reference)
import jax
import jax.numpy as jnp
from jax.experimental import pallas as pl


def kernel(features, adj, W_neigh, W_lin):
    raise NotImplementedError("write your pallas kernel here")


# speedup vs baseline: 1.9111x; 1.9111x over previous
import functools

import jax
import jax.numpy as jnp
from jax.experimental import pallas as pl
from jax.experimental.pallas import tpu as pltpu


def _sage_block(adj_ref, feats_ref, feats_blk_ref, wn_ref, wl_ref, out_ref, *, d):
    adj = adj_ref[...]
    acc = jnp.dot(adj, feats_ref[...], preferred_element_type=jnp.float32)
    deg = jnp.sum(adj, axis=1, keepdims=True) + 1.0
    wl = wl_ref[...]
    wl1 = wl[:, :d]
    wl2 = wl[:, d:]
    wc = jnp.dot(wn_ref[...].T, wl2.T, preferred_element_type=jnp.float32)
    z = jnp.dot(feats_blk_ref[...], wl1.T, preferred_element_type=jnp.float32)
    z = z + jnp.dot(acc, wc, preferred_element_type=jnp.float32) / deg
    out_ref[...] = z


@jax.jit
def kernel(features, adj, W_neigh, W_lin):
    n, d = features.shape
    out = W_lin.shape[0]
    bm = 400
    grid = (n // bm,)
    return pl.pallas_call(
        functools.partial(_sage_block, d=d),
        grid=grid,
        in_specs=[
            pl.BlockSpec((bm, n), lambda i: (i, 0)),
            pl.BlockSpec((n, d), lambda i: (0, 0)),
            pl.BlockSpec((bm, d), lambda i: (i, 0)),
            pl.BlockSpec((d, d), lambda i: (0, 0)),
            pl.BlockSpec((out, 2 * d), lambda i: (0, 0)),
        ],
        out_specs=pl.BlockSpec((bm, out), lambda i: (i, 0)),
        out_shape=jax.ShapeDtypeStruct((n, out), jnp.float32),
        compiler_params=pltpu.CompilerParams(
            dimension_semantics=("arbitrary",),
        ),
    )(adj, features, features, W_neigh, W_lin)
